# Initial kernel scaffold; baseline (speedup 1.0000x reference)
import jax
import jax.numpy as jnp
from jax.experimental import pallas as pl


def kernel(x, edge_index, batch, Wl, bl, Wr, t):
    raise NotImplementedError("write your pallas kernel here")


# speedup vs baseline: 4.1297x; 4.1297x over previous
import functools

import jax
import jax.numpy as jnp
from jax import lax
from jax.experimental import pallas as pl
from jax.experimental.pallas import tpu as pltpu
from jax.experimental.pallas import tpu_sc as plsc

_N = 10000
_F = 128
_B = 64
_L = 8
_E = 320000

_NC = 2
_NS = 16
_NW = _NC * _NS
_CH = 128
_K = 79
_EPAD = _NW * _K * _CH
_NALLOC = 10240
_RPT_Z = _NALLOC // _NS
_RPT_O = _N // _NS
_DUMMY = 10200

_RB = 1000
_GRID = _N // _RB
_NEG = -3.4e38

@functools.cache
def _sc_kernels():
    mesh = plsc.VectorSubcoreMesh(
        core_axis_name="c", subcore_axis_name="s", num_cores=_NC, num_subcores=_NS
    )

    @functools.partial(
        pl.kernel,
        out_type=jax.ShapeDtypeStruct((_NC, _NALLOC, _F), jnp.float32),
        mesh=mesh,
        scratch_types=[
            pltpu.VMEM((_K, _CH), jnp.int32),
            pltpu.VMEM((_K, _CH), jnp.int32),
            pltpu.VMEM((_CH, _F), jnp.float32),
            pltpu.VMEM_SHARED((_NALLOC, _F), jnp.float32),
            pltpu.SemaphoreType.DMA,
        ],
    )
    def _nsum(h_hbm, src_hbm, dst_hbm, out_hbm, src_v, dst_v, rows_v, acc, sem):
        cid = lax.axis_index("c")
        sid = lax.axis_index("s")
        wid = cid * _NS + sid

        def _zb(i, carry):
            rows_v[i // 8, pl.ds((i % 8) * 16, 16)] = jnp.zeros((16,), jnp.float32)
            return carry

        lax.fori_loop(0, (_CH * _F) // 16, _zb, 0)
        for b in range(_RPT_Z // _CH):
            pltpu.sync_copy(rows_v, acc.at[pl.ds(sid * _RPT_Z + b * _CH, _CH)])

        pltpu.sync_copy(src_hbm.at[wid], src_v)
        pltpu.sync_copy(dst_hbm.at[wid], dst_v)
        plsc.subcore_barrier()

        def _body(j, carry):
            pltpu.async_copy(h_hbm.at[src_v.at[j]], rows_v, sem).wait()
            pltpu.sync_copy(rows_v, acc.at[dst_v.at[j]], add=True)
            return carry

        lax.fori_loop(0, _K, _body, 0)
        plsc.subcore_barrier()

        pltpu.sync_copy(
            acc.at[pl.ds(sid * _RPT_Z, _RPT_Z)],
            out_hbm.at[cid, pl.ds(sid * _RPT_Z, _RPT_Z)],
        )

    @functools.partial(
        pl.kernel,
        out_type=jax.ShapeDtypeStruct((_NC, _NALLOC, 8), jnp.float32),
        mesh=mesh,
        scratch_types=[
            pltpu.VMEM((_K, _CH), jnp.int32),
            pltpu.VMEM((_CH, 8), jnp.float32),
            pltpu.VMEM_SHARED((_NALLOC, 8), jnp.float32),
        ],
    )
    def _deg(dst_hbm, ones_hbm, zeros_hbm, out_hbm, dst_v, ones_v, acc):
        cid = lax.axis_index("c")
        sid = lax.axis_index("s")
        wid = cid * _NS + sid

        pltpu.sync_copy(zeros_hbm, acc.at[pl.ds(sid * _RPT_Z, _RPT_Z)])
        pltpu.sync_copy(ones_hbm, ones_v)
        pltpu.sync_copy(dst_hbm.at[wid], dst_v)
        plsc.subcore_barrier()

        def _body(j, carry):
            pltpu.sync_copy(ones_v, acc.at[dst_v.at[j]], add=True)
            return carry

        lax.fori_loop(0, _K, _body, 0)
        plsc.subcore_barrier()

        pltpu.sync_copy(
            acc.at[pl.ds(sid * _RPT_Z, _RPT_Z)],
            out_hbm.at[cid, pl.ds(sid * _RPT_Z, _RPT_Z)],
        )

    return _nsum, _deg


def _sc_neighbor_sum(h, srcp, dstp):
    return _sc_kernels()[0](h, srcp, dstp)


def _sc_degree(dstp, ones8, zeros8):
    return _sc_kernels()[1](dstp, ones8, zeros8)


def _deginv_body(degp_ref, out_ref):
    d = degp_ref[0, :, 0:1] + degp_ref[1, :, 0:1]
    out_ref[...] = 1.0 / jnp.maximum(d, 1.0)


def _deginv(degp):
    return pl.pallas_call(
        _deginv_body,
        grid=(_GRID,),
        in_specs=[pl.BlockSpec((_NC, 1024, 8), lambda i: (0, i, 0))],
        out_specs=pl.BlockSpec((1024, 1), lambda i: (i, 0)),
        out_shape=jax.ShapeDtypeStruct((_NALLOC, 1), jnp.float32),
    )(degp)


def _dense_body(p_ref, dinv_ref, h_ref, wl_ref, b_ref, wr_ref, out_ref):
    mean = (p_ref[0] + p_ref[1]) * dinv_ref[...]
    a = jnp.dot(mean, wl_ref[...], preferred_element_type=jnp.float32)
    c = jnp.dot(h_ref[...], wr_ref[...], preferred_element_type=jnp.float32)
    out_ref[...] = jnp.maximum(a + c + b_ref[...], 0.0)


def _dense(p, dinv, h, wl, b, wr):
    return pl.pallas_call(
        _dense_body,
        grid=(_GRID,),
        in_specs=[
            pl.BlockSpec((_NC, _RB, _F), lambda i: (0, i, 0)),
            pl.BlockSpec((_RB, 1), lambda i: (i, 0)),
            pl.BlockSpec((_RB, _F), lambda i: (i, 0)),
            pl.BlockSpec((_F, _F), lambda i: (0, 0)),
            pl.BlockSpec((1, _F), lambda i: (0, 0)),
            pl.BlockSpec((_F, _F), lambda i: (0, 0)),
        ],
        out_specs=pl.BlockSpec((_RB, _F), lambda i: (i, 0)),
        out_shape=jax.ShapeDtypeStruct((_N, _F), jnp.float32),
    )(p, dinv, h, wl, b, wr)


def _segmax_body(h_ref, bc_ref, t_ref, zmax_ref):
    i = pl.program_id(0)

    @pl.when(i == 0)
    def _():
        zmax_ref[...] = jnp.full((_B, _F), _NEG, jnp.float32)

    z = h_ref[...] * t_ref[...]
    bc = bc_ref[...]
    lo = bc_ref[0, 0]
    hi = bc_ref[_RB - 1, 0]

    def _bstep(b, carry):
        mx = jnp.max(jnp.where(bc == b, z, _NEG), axis=0, keepdims=True)
        zmax_ref[pl.ds(b, 1), :] = jnp.maximum(zmax_ref[pl.ds(b, 1), :], mx)
        return carry

    lax.fori_loop(lo, hi + 1, _bstep, 0)

    @pl.when(i == _GRID - 1)
    def _():
        zm = zmax_ref[...]
        zmax_ref[...] = jnp.where(zm > _NEG * 0.5, zm, 0.0)


def _segmax(h, bc, t11):
    return pl.pallas_call(
        _segmax_body,
        grid=(_GRID,),
        in_specs=[
            pl.BlockSpec((_RB, _F), lambda i: (i, 0)),
            pl.BlockSpec((_RB, 1), lambda i: (i, 0)),
            pl.BlockSpec((1, 1), lambda i: (0, 0)),
        ],
        out_specs=pl.BlockSpec((_B, _F), lambda i: (0, 0)),
        out_shape=jax.ShapeDtypeStruct((_B, _F), jnp.float32),
    )(h, bc, t11)


def _softagg_body(h_ref, bc_ref, zmax_ref, t_ref, out_ref, den_ref, num_ref):
    i = pl.program_id(0)

    @pl.when(i == 0)
    def _():
        den_ref[...] = jnp.zeros((_B, _F), jnp.float32)
        num_ref[...] = jnp.zeros((_B, _F), jnp.float32)

    h = h_ref[...]
    z = h * t_ref[...]
    m = (bc_ref[...] == lax.broadcasted_iota(jnp.int32, (_RB, _B), 1)).astype(
        jnp.float32
    )
    zr = jnp.dot(m, zmax_ref[...], preferred_element_type=jnp.float32)
    ez = jnp.exp(z - zr)
    dn = (((0,), (0,)), ((), ()))
    den_ref[...] += lax.dot_general(m, ez, dn, preferred_element_type=jnp.float32)
    num_ref[...] += lax.dot_general(m, h * ez, dn, preferred_element_type=jnp.float32)

    @pl.when(i == _GRID - 1)
    def _():
        out_ref[...] = num_ref[...] / jnp.maximum(den_ref[...], 1e-16)


def _softagg(h, bc, zmax, t11):
    return pl.pallas_call(
        _softagg_body,
        grid=(_GRID,),
        in_specs=[
            pl.BlockSpec((_RB, _F), lambda i: (i, 0)),
            pl.BlockSpec((_RB, 1), lambda i: (i, 0)),
            pl.BlockSpec((_B, _F), lambda i: (0, 0)),
            pl.BlockSpec((1, 1), lambda i: (0, 0)),
        ],
        out_specs=pl.BlockSpec((_B, _F), lambda i: (0, 0)),
        out_shape=jax.ShapeDtypeStruct((_B, _F), jnp.float32),
        scratch_shapes=[
            pltpu.VMEM((_B, _F), jnp.float32),
            pltpu.VMEM((_B, _F), jnp.float32),
        ],
    )(h, bc, zmax, t11)


def kernel(x, edge_index, batch, Wl, bl, Wr, t):
    src = edge_index[0]
    dst = edge_index[1]
    pad = _EPAD - _E
    srcp = jnp.concatenate([src, jnp.zeros((pad,), jnp.int32)]).reshape(_NW, _K, _CH)
    dstp = jnp.concatenate([dst, jnp.full((pad,), _DUMMY, jnp.int32)]).reshape(
        _NW, _K, _CH
    )

    ones8 = jnp.ones((_CH, 8), jnp.float32)
    zeros8 = jnp.zeros((_RPT_Z, 8), jnp.float32)
    degp = _sc_degree(dstp, ones8, zeros8)
    dinv = _deginv(degp)

    t11 = t.reshape(1, 1)
    h = x
    for i in range(_L):
        p = _sc_neighbor_sum(h, srcp, dstp)
        h = _dense(p, dinv, h, Wl[i], bl[i].reshape(1, _F), Wr[i])

    bc = batch.reshape(_N, 1)
    zmax = _segmax(h, bc, t11)
    return _softagg(h, bc, zmax, t11)


# speedup vs baseline: 4.6819x; 1.1337x over previous
import functools

import jax
import jax.numpy as jnp
from jax import lax
from jax.experimental import pallas as pl
from jax.experimental.pallas import tpu as pltpu
from jax.experimental.pallas import tpu_sc as plsc

_N = 10000
_F = 128
_B = 64
_L = 8
_E = 320000

_NC = 2
_NS = 16
_NW = _NC * _NS
_CH = 128
_K = 79
_EPAD = _NW * _K * _CH
_NALLOC = 10240
_RPT_Z = _NALLOC // _NS
_RPT_O = _N // _NS
_DUMMY = 10200

_RB = 1000
_GRID = _N // _RB
_NEG = -3.4e38

@functools.cache
def _sc_kernels():
    mesh = plsc.VectorSubcoreMesh(
        core_axis_name="c", subcore_axis_name="s", num_cores=_NC, num_subcores=_NS
    )

    @functools.partial(
        pl.kernel,
        out_type=jax.ShapeDtypeStruct((_NC, _NALLOC, _F), jnp.float32),
        mesh=mesh,
        scratch_types=[
            pltpu.VMEM((_K, _CH), jnp.int32),
            pltpu.VMEM((1, _CH), jnp.int32),
            pltpu.VMEM((1, _CH), jnp.int32),
            pltpu.VMEM((_CH, _F), jnp.float32),
            pltpu.VMEM((_CH, _F), jnp.float32),
            pltpu.VMEM_SHARED((_NALLOC, _F), jnp.float32),
            pltpu.SemaphoreType.DMA,
            pltpu.SemaphoreType.DMA,
            pltpu.SemaphoreType.DMA,
            pltpu.SemaphoreType.DMA,
        ],
    )
    def _nsum(
        h_hbm, src_hbm, dst_hbm, out_hbm,
        src_v, dbuf_v, dbuf_w, rows_v, rows_w, acc,
        sem, sem2, semd, semd2,
    ):
        cid = lax.axis_index("c")
        sid = lax.axis_index("s")
        wid = cid * _NS + sid

        def _zb(i, carry):
            rows_v[i // 8, pl.ds((i % 8) * 16, 16)] = jnp.zeros((16,), jnp.float32)
            return carry

        lax.fori_loop(0, (_CH * _F) // 16, _zb, 0)
        for b in range(_RPT_Z // _CH):
            pltpu.sync_copy(rows_v, acc.at[pl.ds(sid * _RPT_Z + b * _CH, _CH)])

        pltpu.sync_copy(src_hbm.at[wid], src_v)
        plsc.subcore_barrier()

        pltpu.sync_copy(dst_hbm.at[wid, pl.ds(0, 1)], dbuf_v)
        pltpu.async_copy(h_hbm.at[src_v.at[0]], rows_v, sem).wait()

        def _body(jj, carry):
            j = 2 * jj
            cp1 = pltpu.async_copy(h_hbm.at[src_v.at[j + 1]], rows_w, sem2)
            cpd1 = pltpu.async_copy(dst_hbm.at[wid, pl.ds(j + 1, 1)], dbuf_w, semd2)
            pltpu.sync_copy(rows_v, acc.at[dbuf_v.at[0]], add=True)
            cp1.wait()
            cpd1.wait()
            cp2 = pltpu.async_copy(h_hbm.at[src_v.at[j + 2]], rows_v, sem)
            cpd2 = pltpu.async_copy(dst_hbm.at[wid, pl.ds(j + 2, 1)], dbuf_v, semd)
            pltpu.sync_copy(rows_w, acc.at[dbuf_w.at[0]], add=True)
            cp2.wait()
            cpd2.wait()
            return carry

        lax.fori_loop(0, (_K - 1) // 2, _body, 0)
        pltpu.sync_copy(rows_v, acc.at[dbuf_v.at[0]], add=True)
        plsc.subcore_barrier()

        pltpu.sync_copy(
            acc.at[pl.ds(sid * _RPT_Z, _RPT_Z)],
            out_hbm.at[cid, pl.ds(sid * _RPT_Z, _RPT_Z)],
        )

    @functools.partial(
        pl.kernel,
        out_type=jax.ShapeDtypeStruct((_NC, _NALLOC, 8), jnp.float32),
        mesh=mesh,
        scratch_types=[
            pltpu.VMEM((_K, _CH), jnp.int32),
            pltpu.VMEM((_CH, 8), jnp.float32),
            pltpu.VMEM_SHARED((_NALLOC, 8), jnp.float32),
        ],
    )
    def _deg(dst_hbm, ones_hbm, zeros_hbm, out_hbm, dst_v, ones_v, acc):
        cid = lax.axis_index("c")
        sid = lax.axis_index("s")
        wid = cid * _NS + sid

        pltpu.sync_copy(zeros_hbm, acc.at[pl.ds(sid * _RPT_Z, _RPT_Z)])
        pltpu.sync_copy(ones_hbm, ones_v)
        pltpu.sync_copy(dst_hbm.at[wid], dst_v)
        plsc.subcore_barrier()

        def _body(j, carry):
            pltpu.sync_copy(ones_v, acc.at[dst_v.at[j]], add=True)
            return carry

        lax.fori_loop(0, _K, _body, 0)
        plsc.subcore_barrier()

        pltpu.sync_copy(
            acc.at[pl.ds(sid * _RPT_Z, _RPT_Z)],
            out_hbm.at[cid, pl.ds(sid * _RPT_Z, _RPT_Z)],
        )

    return _nsum, _deg


def _sc_neighbor_sum(h, srcp, dstp):
    return _sc_kernels()[0](h, srcp, dstp)


def _sc_degree(dstp, ones8, zeros8):
    return _sc_kernels()[1](dstp, ones8, zeros8)


def _deginv_body(degp_ref, out_ref):
    d = degp_ref[0, :, 0:1] + degp_ref[1, :, 0:1]
    out_ref[...] = 1.0 / jnp.maximum(d, 1.0)


def _deginv(degp):
    return pl.pallas_call(
        _deginv_body,
        grid=(_GRID,),
        in_specs=[pl.BlockSpec((_NC, 1024, 8), lambda i: (0, i, 0))],
        out_specs=pl.BlockSpec((1024, 1), lambda i: (i, 0)),
        out_shape=jax.ShapeDtypeStruct((_NALLOC, 1), jnp.float32),
    )(degp)


def _dense_body(p_ref, dinv_ref, h_ref, wl_ref, b_ref, wr_ref, out_ref):
    mean = (p_ref[0] + p_ref[1]) * dinv_ref[...]
    a = jnp.dot(mean, wl_ref[...], preferred_element_type=jnp.float32)
    c = jnp.dot(h_ref[...], wr_ref[...], preferred_element_type=jnp.float32)
    out_ref[...] = jnp.maximum(a + c + b_ref[...], 0.0)


def _dense(p, dinv, h, wl, b, wr):
    return pl.pallas_call(
        _dense_body,
        grid=(_GRID,),
        in_specs=[
            pl.BlockSpec((_NC, _RB, _F), lambda i: (0, i, 0)),
            pl.BlockSpec((_RB, 1), lambda i: (i, 0)),
            pl.BlockSpec((_RB, _F), lambda i: (i, 0)),
            pl.BlockSpec((_F, _F), lambda i: (0, 0)),
            pl.BlockSpec((1, _F), lambda i: (0, 0)),
            pl.BlockSpec((_F, _F), lambda i: (0, 0)),
        ],
        out_specs=pl.BlockSpec((_RB, _F), lambda i: (i, 0)),
        out_shape=jax.ShapeDtypeStruct((_N, _F), jnp.float32),
    )(p, dinv, h, wl, b, wr)


def _segmax_body(h_ref, bc_ref, t_ref, zmax_ref):
    i = pl.program_id(0)

    @pl.when(i == 0)
    def _():
        zmax_ref[...] = jnp.full((_B, _F), _NEG, jnp.float32)

    z = h_ref[...] * t_ref[...]
    bc = bc_ref[...]
    lo = bc_ref[0, 0]
    hi = bc_ref[_RB - 1, 0]

    def _bstep(b, carry):
        mx = jnp.max(jnp.where(bc == b, z, _NEG), axis=0, keepdims=True)
        zmax_ref[pl.ds(b, 1), :] = jnp.maximum(zmax_ref[pl.ds(b, 1), :], mx)
        return carry

    lax.fori_loop(lo, hi + 1, _bstep, 0)

    @pl.when(i == _GRID - 1)
    def _():
        zm = zmax_ref[...]
        zmax_ref[...] = jnp.where(zm > _NEG * 0.5, zm, 0.0)


def _segmax(h, bc, t11):
    return pl.pallas_call(
        _segmax_body,
        grid=(_GRID,),
        in_specs=[
            pl.BlockSpec((_RB, _F), lambda i: (i, 0)),
            pl.BlockSpec((_RB, 1), lambda i: (i, 0)),
            pl.BlockSpec((1, 1), lambda i: (0, 0)),
        ],
        out_specs=pl.BlockSpec((_B, _F), lambda i: (0, 0)),
        out_shape=jax.ShapeDtypeStruct((_B, _F), jnp.float32),
    )(h, bc, t11)


def _softagg_body(h_ref, bc_ref, zmax_ref, t_ref, out_ref, den_ref, num_ref):
    i = pl.program_id(0)

    @pl.when(i == 0)
    def _():
        den_ref[...] = jnp.zeros((_B, _F), jnp.float32)
        num_ref[...] = jnp.zeros((_B, _F), jnp.float32)

    h = h_ref[...]
    z = h * t_ref[...]
    m = (bc_ref[...] == lax.broadcasted_iota(jnp.int32, (_RB, _B), 1)).astype(
        jnp.float32
    )
    zr = jnp.dot(m, zmax_ref[...], preferred_element_type=jnp.float32)
    ez = jnp.exp(z - zr)
    dn = (((0,), (0,)), ((), ()))
    den_ref[...] += lax.dot_general(m, ez, dn, preferred_element_type=jnp.float32)
    num_ref[...] += lax.dot_general(m, h * ez, dn, preferred_element_type=jnp.float32)

    @pl.when(i == _GRID - 1)
    def _():
        out_ref[...] = num_ref[...] / jnp.maximum(den_ref[...], 1e-16)


def _softagg(h, bc, zmax, t11):
    return pl.pallas_call(
        _softagg_body,
        grid=(_GRID,),
        in_specs=[
            pl.BlockSpec((_RB, _F), lambda i: (i, 0)),
            pl.BlockSpec((_RB, 1), lambda i: (i, 0)),
            pl.BlockSpec((_B, _F), lambda i: (0, 0)),
            pl.BlockSpec((1, 1), lambda i: (0, 0)),
        ],
        out_specs=pl.BlockSpec((_B, _F), lambda i: (0, 0)),
        out_shape=jax.ShapeDtypeStruct((_B, _F), jnp.float32),
        scratch_shapes=[
            pltpu.VMEM((_B, _F), jnp.float32),
            pltpu.VMEM((_B, _F), jnp.float32),
        ],
    )(h, bc, zmax, t11)


def kernel(x, edge_index, batch, Wl, bl, Wr, t):
    src = edge_index[0]
    dst = edge_index[1]
    pad = _EPAD - _E
    srcp = jnp.concatenate([src, jnp.zeros((pad,), jnp.int32)]).reshape(_NW, _K, _CH)
    dstp = jnp.concatenate([dst, jnp.full((pad,), _DUMMY, jnp.int32)]).reshape(
        _NW, _K, _CH
    )

    ones8 = jnp.ones((_CH, 8), jnp.float32)
    zeros8 = jnp.zeros((_RPT_Z, 8), jnp.float32)
    degp = _sc_degree(dstp, ones8, zeros8)
    dinv = _deginv(degp)

    t11 = t.reshape(1, 1)
    h = x
    for i in range(_L):
        p = _sc_neighbor_sum(h, srcp, dstp)
        h = _dense(p, dinv, h, Wl[i], bl[i].reshape(1, _F), Wr[i])

    bc = batch.reshape(_N, 1)
    zmax = _segmax(h, bc, t11)
    return _softagg(h, bc, zmax, t11)


# speedup vs baseline: 4.9265x; 1.0522x over previous
import functools

import jax
import jax.numpy as jnp
from jax import lax
from jax.experimental import pallas as pl
from jax.experimental.pallas import tpu as pltpu
from jax.experimental.pallas import tpu_sc as plsc

_N = 10000
_F = 128
_B = 64
_L = 8
_E = 320000

_NC = 2
_NS = 16
_NW = _NC * _NS
_CH = 128
_K = 79
_EPAD = _NW * _K * _CH
_NALLOC = 10112
_RPT_Z = _NALLOC // _NS
_DUMMY = 10104

_RB = 1000
_GRID = _N // _RB
_NEG = -3.4e38


@functools.cache
def _sc_kernels():
    mesh = plsc.VectorSubcoreMesh(
        core_axis_name="c", subcore_axis_name="s", num_cores=_NC, num_subcores=_NS
    )

    @functools.partial(
        pl.kernel,
        out_type=jax.ShapeDtypeStruct((_NC, _NALLOC, _F), jnp.float32),
        mesh=mesh,
        scratch_types=[
            [pltpu.VMEM((1, _CH), jnp.int32)] * 3,
            [pltpu.VMEM((1, _CH), jnp.int32)] * 3,
            [pltpu.VMEM((_CH, _F), jnp.float32)] * 3,
            pltpu.VMEM_SHARED((_NALLOC, _F), jnp.float32),
            [pltpu.SemaphoreType.DMA] * 3,
            [pltpu.SemaphoreType.DMA] * 3,
            [pltpu.SemaphoreType.DMA] * 3,
        ],
    )
    def _nsum(h_hbm, src_hbm, dst_hbm, out_hbm, sidx, didx, rows, acc, semg, semsi, semdi):
        cid = lax.axis_index("c")
        sid = lax.axis_index("s")
        wid = cid * _NS + sid

        def _zb(i, carry):
            rows[0][i // 8, pl.ds((i % 8) * 16, 16)] = jnp.zeros((16,), jnp.float32)
            return carry

        lax.fori_loop(0, (_CH * _F) // 16, _zb, 0)
        for b in range(4):
            pltpu.sync_copy(rows[0], acc.at[pl.ds(sid * _RPT_Z + b * _CH, _CH)])
        pltpu.sync_copy(
            rows[0].at[pl.ds(0, _RPT_Z - 4 * _CH)],
            acc.at[pl.ds(sid * _RPT_Z + 4 * _CH, _RPT_Z - 4 * _CH)],
        )
        plsc.subcore_barrier()

        def _load_idx(j, s):
            pltpu.async_copy(src_hbm.at[wid, pl.ds(j, 1)], sidx[s], semsi[s])
            pltpu.async_copy(dst_hbm.at[wid, pl.ds(j, 1)], didx[s], semdi[s])

        def _wait_sidx(j, s):
            pltpu.make_async_copy(src_hbm.at[wid, pl.ds(j, 1)], sidx[s], semsi[s]).wait()

        def _wait_didx(j, s):
            pltpu.make_async_copy(dst_hbm.at[wid, pl.ds(j, 1)], didx[s], semdi[s]).wait()

        def _start_gather(s):
            pltpu.async_copy(h_hbm.at[sidx[s].at[0]], rows[s], semg[s])

        def _wait_gather(s):
            pltpu.make_async_copy(h_hbm.at[sidx[s].at[0]], rows[s], semg[s]).wait()

        for s in range(3):
            _load_idx(s, s)
        for s in range(2):
            _wait_sidx(s, s)
            _start_gather(s)

        def _step(j, s, prefetch=True):
            s2 = (s + 2) % 3
            _wait_gather(s)
            _wait_sidx(j + 2, s2)
            _start_gather(s2)
            _wait_didx(j, s)
            pltpu.sync_copy(rows[s], acc.at[didx[s].at[0]], add=True)
            if prefetch:
                _load_idx(j + 3, s)

        def _body(jj, carry):
            j = 3 * jj
            _step(j, 0)
            _step(j + 1, 1)
            _step(j + 2, 2)
            return carry

        lax.fori_loop(0, (_K - 4) // 3, _body, 0)
        _step(_K - 4, 0)
        _step(_K - 3, 1, prefetch=False)
        for j, s in ((_K - 2, 2), (_K - 1, 0)):
            _wait_gather(s)
            _wait_didx(j, s)
            pltpu.sync_copy(rows[s], acc.at[didx[s].at[0]], add=True)
        plsc.subcore_barrier()

        pltpu.sync_copy(
            acc.at[pl.ds(sid * _RPT_Z, _RPT_Z)],
            out_hbm.at[cid, pl.ds(sid * _RPT_Z, _RPT_Z)],
        )

    @functools.partial(
        pl.kernel,
        out_type=jax.ShapeDtypeStruct((_NC, _NALLOC, 8), jnp.float32),
        mesh=mesh,
        scratch_types=[
            pltpu.VMEM((_K, _CH), jnp.int32),
            pltpu.VMEM((_CH, 8), jnp.float32),
            pltpu.VMEM_SHARED((_NALLOC, 8), jnp.float32),
            [pltpu.SemaphoreType.DMA] * 4,
        ],
    )
    def _deg(dst_hbm, ones_hbm, zeros_hbm, out_hbm, dst_v, ones_v, acc, sems):
        cid = lax.axis_index("c")
        sid = lax.axis_index("s")
        wid = cid * _NS + sid

        pltpu.sync_copy(zeros_hbm, acc.at[pl.ds(sid * _RPT_Z, _RPT_Z)])
        pltpu.sync_copy(ones_hbm, ones_v)
        pltpu.sync_copy(dst_hbm.at[wid], dst_v)
        plsc.subcore_barrier()

        def _body(jj, carry):
            j = 4 * jj
            cps = [
                pltpu.async_copy(ones_v, acc.at[dst_v.at[j + u]], sems[u], add=True)
                for u in range(4)
            ]
            for cp in cps:
                cp.wait()
            return carry

        lax.fori_loop(0, _K // 4, _body, 0)
        cps = [
            pltpu.async_copy(ones_v, acc.at[dst_v.at[_K - 3 + u]], sems[u], add=True)
            for u in range(3)
        ]
        for cp in cps:
            cp.wait()
        plsc.subcore_barrier()

        pltpu.sync_copy(
            acc.at[pl.ds(sid * _RPT_Z, _RPT_Z)],
            out_hbm.at[cid, pl.ds(sid * _RPT_Z, _RPT_Z)],
        )

    return _nsum, _deg


def _sc_neighbor_sum(h, srcp, dstp):
    return _sc_kernels()[0](h, srcp, dstp)


def _sc_degree(dstp, ones8, zeros8):
    return _sc_kernels()[1](dstp, ones8, zeros8)


def _deginv_body(degp_ref, out_ref):
    d = degp_ref[0, :, 0:1] + degp_ref[1, :, 0:1]
    out_ref[...] = 1.0 / jnp.maximum(d, 1.0)


def _deginv(degp):
    return pl.pallas_call(
        _deginv_body,
        grid=(_GRID,),
        in_specs=[pl.BlockSpec((_NC, 1024, 8), lambda i: (0, i, 0))],
        out_specs=pl.BlockSpec((1024, 1), lambda i: (i, 0)),
        out_shape=jax.ShapeDtypeStruct((10240, 1), jnp.float32),
    )(degp)


def _dense_body(p_ref, dinv_ref, h_ref, wl_ref, b_ref, wr_ref, out_ref):
    mean = (p_ref[0] + p_ref[1]) * dinv_ref[...]
    a = jnp.dot(mean, wl_ref[...], preferred_element_type=jnp.float32)
    c = jnp.dot(h_ref[...], wr_ref[...], preferred_element_type=jnp.float32)
    out_ref[...] = jnp.maximum(a + c + b_ref[...], 0.0)


def _dense(p, dinv, h, wl, b, wr):
    return pl.pallas_call(
        _dense_body,
        grid=(_GRID,),
        in_specs=[
            pl.BlockSpec((_NC, _RB, _F), lambda i: (0, i, 0)),
            pl.BlockSpec((_RB, 1), lambda i: (i, 0)),
            pl.BlockSpec((_RB, _F), lambda i: (i, 0)),
            pl.BlockSpec((_F, _F), lambda i: (0, 0)),
            pl.BlockSpec((1, _F), lambda i: (0, 0)),
            pl.BlockSpec((_F, _F), lambda i: (0, 0)),
        ],
        out_specs=pl.BlockSpec((_RB, _F), lambda i: (i, 0)),
        out_shape=jax.ShapeDtypeStruct((_N, _F), jnp.float32),
    )(p, dinv, h, wl, b, wr)


def _segmax_body(h_ref, bc_ref, t_ref, zmax_ref):
    i = pl.program_id(0)

    @pl.when(i == 0)
    def _():
        zmax_ref[...] = jnp.full((_B, _F), _NEG, jnp.float32)

    z = h_ref[...] * t_ref[...]
    bc = bc_ref[...]
    lo = bc_ref[0, 0]
    hi = bc_ref[_RB - 1, 0]

    def _bstep(b, carry):
        mx = jnp.max(jnp.where(bc == b, z, _NEG), axis=0, keepdims=True)
        zmax_ref[pl.ds(b, 1), :] = jnp.maximum(zmax_ref[pl.ds(b, 1), :], mx)
        return carry

    lax.fori_loop(lo, hi + 1, _bstep, 0)

    @pl.when(i == _GRID - 1)
    def _():
        zm = zmax_ref[...]
        zmax_ref[...] = jnp.where(zm > _NEG * 0.5, zm, 0.0)


def _segmax(h, bc, t11):
    return pl.pallas_call(
        _segmax_body,
        grid=(_GRID,),
        in_specs=[
            pl.BlockSpec((_RB, _F), lambda i: (i, 0)),
            pl.BlockSpec((_RB, 1), lambda i: (i, 0)),
            pl.BlockSpec((1, 1), lambda i: (0, 0)),
        ],
        out_specs=pl.BlockSpec((_B, _F), lambda i: (0, 0)),
        out_shape=jax.ShapeDtypeStruct((_B, _F), jnp.float32),
    )(h, bc, t11)


def _softagg_body(h_ref, bc_ref, zmax_ref, t_ref, out_ref, den_ref, num_ref):
    i = pl.program_id(0)

    @pl.when(i == 0)
    def _():
        den_ref[...] = jnp.zeros((_B, _F), jnp.float32)
        num_ref[...] = jnp.zeros((_B, _F), jnp.float32)

    h = h_ref[...]
    z = h * t_ref[...]
    m = (bc_ref[...] == lax.broadcasted_iota(jnp.int32, (_RB, _B), 1)).astype(
        jnp.float32
    )
    zr = jnp.dot(m, zmax_ref[...], preferred_element_type=jnp.float32)
    ez = jnp.exp(z - zr)
    dn = (((0,), (0,)), ((), ()))
    den_ref[...] += lax.dot_general(m, ez, dn, preferred_element_type=jnp.float32)
    num_ref[...] += lax.dot_general(m, h * ez, dn, preferred_element_type=jnp.float32)

    @pl.when(i == _GRID - 1)
    def _():
        out_ref[...] = num_ref[...] / jnp.maximum(den_ref[...], 1e-16)


def _softagg(h, bc, zmax, t11):
    return pl.pallas_call(
        _softagg_body,
        grid=(_GRID,),
        in_specs=[
            pl.BlockSpec((_RB, _F), lambda i: (i, 0)),
            pl.BlockSpec((_RB, 1), lambda i: (i, 0)),
            pl.BlockSpec((_B, _F), lambda i: (0, 0)),
            pl.BlockSpec((1, 1), lambda i: (0, 0)),
        ],
        out_specs=pl.BlockSpec((_B, _F), lambda i: (0, 0)),
        out_shape=jax.ShapeDtypeStruct((_B, _F), jnp.float32),
        scratch_shapes=[
            pltpu.VMEM((_B, _F), jnp.float32),
            pltpu.VMEM((_B, _F), jnp.float32),
        ],
    )(h, bc, zmax, t11)


def kernel(x, edge_index, batch, Wl, bl, Wr, t):
    src = edge_index[0]
    dst = edge_index[1]
    pad = _EPAD - _E
    srcp = jnp.concatenate([src, jnp.zeros((pad,), jnp.int32)]).reshape(_NW, _K, _CH)
    dstp = jnp.concatenate([dst, jnp.full((pad,), _DUMMY, jnp.int32)]).reshape(
        _NW, _K, _CH
    )

    ones8 = jnp.ones((_CH, 8), jnp.float32)
    zeros8 = jnp.zeros((_RPT_Z, 8), jnp.float32)
    degp = _sc_degree(dstp, ones8, zeros8)
    dinv = _deginv(degp)

    t11 = t.reshape(1, 1)
    h = x
    for i in range(_L):
        p = _sc_neighbor_sum(h, srcp, dstp)
        h = _dense(p, dinv, h, Wl[i], bl[i].reshape(1, _F), Wr[i])

    bc = batch.reshape(_N, 1)
    zmax = _segmax(h, bc, t11)
    return _softagg(h, bc, zmax, t11)
